# Initial kernel scaffold; baseline (speedup 1.0000x reference)
import jax
import jax.numpy as jnp
from jax.experimental import pallas as pl


def kernel(x, A, mask, extra, coord, weight, root, bias):
    raise NotImplementedError("write your pallas kernel here")


# speedup vs baseline: 1.9781x; 1.9781x over previous
import jax
import jax.numpy as jnp
from jax.experimental import pallas as pl

_K = 25
_KS = 5
_TILE = 512


def _body(coord_ref, mask_ref, x_ref, w_ref, root_ref, bias_ref, out_ref):
    x = x_ref[...]
    xb = x.astype(jnp.bfloat16)
    acc = jnp.dot(xb, root_ref[...], preferred_element_type=jnp.float32)

    c = coord_ref[...]
    v = c * jnp.float32(_KS - 1)
    bot = jnp.floor(v)
    frac = v - bot
    boti = bot.astype(jnp.int32)
    f0 = frac[:, 0:1]
    f1 = frac[:, 1:2]
    b0 = boti[:, 0:1]
    b1 = boti[:, 1:2]

    wis = []
    bas = []
    for s in range(4):
        k0 = s % 2
        k1 = s // 2
        wi = jnp.mod(b0 + k0, _KS) + _KS * jnp.mod(b1 + k1, _KS)
        bs = (f0 if k0 else 1.0 - f0) * (f1 if k1 else 1.0 - f1)
        nan = jnp.isnan(bs)
        wis.append(jnp.where(nan, 0, wi))
        bas.append(jnp.where(nan, 0.0, bs))

    for k in range(_K):
        sk = jnp.zeros_like(f0)
        for s in range(4):
            sk = sk + jnp.where(wis[s] == k, bas[s], 0.0)
        xs = (x * sk).astype(jnp.bfloat16)
        acc = acc + jnp.dot(xs, w_ref[k], preferred_element_type=jnp.float32)

    out = (acc + bias_ref[...]) * mask_ref[...]
    out_ref[...] = out


def kernel(x, A, mask, extra, coord, weight, root, bias):
    Bq, Nq, F = x.shape
    E = Bq * Nq
    Ep = ((E + _TILE - 1) // _TILE) * _TILE
    pad = Ep - E

    x2 = jnp.pad(x.reshape(E, F), ((0, pad), (0, 0)))
    coord2 = jnp.pad(coord.reshape(E, 2), ((0, pad), (0, 0)))
    mask2 = jnp.pad(mask.reshape(E, 1), ((0, pad), (0, 0)))
    wb = weight.astype(jnp.bfloat16)
    rb = root.astype(jnp.bfloat16)

    out = pl.pallas_call(
        _body,
        grid=(Ep // _TILE,),
        in_specs=[
            pl.BlockSpec((_TILE, 2), lambda i: (i, 0)),
            pl.BlockSpec((_TILE, 1), lambda i: (i, 0)),
            pl.BlockSpec((_TILE, F), lambda i: (i, 0)),
            pl.BlockSpec((_K, F, F), lambda i: (0, 0, 0)),
            pl.BlockSpec((F, F), lambda i: (0, 0)),
            pl.BlockSpec((1, F), lambda i: (0, 0)),
        ],
        out_specs=pl.BlockSpec((_TILE, F), lambda i: (i, 0)),
        out_shape=jax.ShapeDtypeStruct((Ep, F), jnp.float32),
    )(coord2, mask2, x2, wb, rb, bias.reshape(1, F))

    return out[:E].reshape(Bq, Nq, F)
